# baseline (speedup 1.0000x reference)
import jax
import jax.numpy as jnp
from jax import lax
from jax.experimental import pallas as pl
from jax.experimental.pallas import tpu as pltpu
from jax.experimental.pallas import tpu_sc as plsc

NC = 2
NS = 16
NW = NC * NS
L = 16

DIM = 64
N_ROWS = 16384
N_COLS = 50
N_PAIRS = N_ROWS * N_COLS
C = 128
BLOCKS_PER_TILE = N_ROWS // C // NW
NCHUNK = N_COLS * BLOCKS_PER_TILE
GROUPS = C // L

NBUF = 2


def _sc_kernel(idx_hbm, emb_hbm, out_hbm,
               idx_v, rows_a0, rows_a1, rows_b0, rows_b1,
               out_v, stage_v, sem0, sem1):
    wid = lax.axis_index("s") * NC + lax.axis_index("c")
    lane = jnp.arange(L, dtype=jnp.int32)
    sems = [sem0, sem1]
    rows_as = [rows_a0, rows_a1]
    rows_bs = [rows_b0, rows_b1]

    def chunk_offsets(h):
        c = h // BLOCKS_PER_TILE
        m1 = wid * BLOCKS_PER_TILE + h % BLOCKS_PER_TILE
        return c * (2 * N_ROWS) + m1 * (2 * C), c * N_ROWS + m1 * C

    def fire_chunk(h, b):
        idx0, _ = chunk_offsets(h)
        pltpu.sync_copy(idx_hbm.at[pl.ds(idx0, 2 * C)],
                        idx_v.at[pl.ds(b * 2 * C, 2 * C)])
        pltpu.async_copy(
            emb_hbm.at[idx_v.at[pl.ds(b * 2 * C, C)]], rows_as[b], sems[b]
        )
        pltpu.async_copy(
            emb_hbm.at[idx_v.at[pl.ds(b * 2 * C + C, C)]], rows_bs[b], sems[b]
        )

    def wait_chunk(b):
        pltpu.make_async_copy(emb_hbm.at[pl.ds(0, C)], rows_as[b], sems[b]).wait()
        pltpu.make_async_copy(emb_hbm.at[pl.ds(0, C)], rows_bs[b], sems[b]).wait()

    def compute_chunk(h, b):
        ra = rows_as[b]
        rb = rows_bs[b]

        @pl.loop(0, GROUPS)
        def _group(t):
            for k in range(L):
                p = t * L + k
                z = jnp.zeros((L,), jnp.float32)
                for q in range(DIM // L):
                    a = ra[p, pl.ds(q * L, L)]
                    bb = rb[p, pl.ds(q * L, L)]
                    r = jnp.maximum(a - bb, 0.0)
                    z = z + r * r
                stage_v[pl.ds(k * L, L)] = z
            vec = jnp.zeros((L,), jnp.float32)
            for d in range(L):
                vec = vec + plsc.load_gather(stage_v, [lane * L + d])
            out_v[pl.ds(t * L, L)] = -vec

        _, out0 = chunk_offsets(h)
        pltpu.sync_copy(out_v, out_hbm.at[pl.ds(out0, C)])

    fire_chunk(0, 0)

    @pl.loop(0, NCHUNK // NBUF)
    def _outer(gg):
        for b in range(NBUF):
            h = gg * NBUF + b

            @pl.when(h + 1 < NCHUNK)
            def _fire_next():
                fire_chunk(h + 1, (b + 1) % NBUF)

            wait_chunk(b)
            compute_chunk(h, b)


@jax.jit
def kernel(idxs, emb):
    idx32 = idxs.astype(jnp.int32)
    idx_flat = (
        idx32.transpose(1, 2, 0)
        .reshape(N_COLS, 2, N_ROWS // C, C)
        .transpose(0, 2, 1, 3)
        .reshape(-1)
    )
    sel = jnp.concatenate(
        [jnp.eye(DIM, dtype=jnp.float32),
         jnp.zeros((DIM, 128 - DIM), jnp.float32)], axis=1
    )
    emb_p = jax.lax.dot(emb, sel, precision=jax.lax.Precision.HIGHEST)
    mesh = plsc.VectorSubcoreMesh(
        core_axis_name="c", subcore_axis_name="s", num_cores=NC, num_subcores=NS
    )
    out = pl.kernel(
        _sc_kernel,
        out_type=jax.ShapeDtypeStruct((N_PAIRS,), jnp.float32),
        mesh=mesh,
        scratch_types=[
            pltpu.VMEM((NBUF * 2 * C,), jnp.int32),
            pltpu.VMEM((C, 128), jnp.float32),
            pltpu.VMEM((C, 128), jnp.float32),
            pltpu.VMEM((C, 128), jnp.float32),
            pltpu.VMEM((C, 128), jnp.float32),
            pltpu.VMEM((C,), jnp.float32),
            pltpu.VMEM((L * L,), jnp.float32),
            pltpu.SemaphoreType.DMA,
            pltpu.SemaphoreType.DMA,
        ],
        compiler_params=pltpu.CompilerParams(needs_layout_passes=False),
    )(idx_flat, emb_p)
    return out.reshape(N_COLS, N_ROWS).T


# speedup vs baseline: 1.3802x; 1.3802x over previous
import jax
import jax.numpy as jnp
from jax import lax
from jax.experimental import pallas as pl
from jax.experimental.pallas import tpu as pltpu
from jax.experimental.pallas import tpu_sc as plsc

NC = 2
NS = 16
NW = NC * NS
L = 16

DIM = 64
N_ROWS = 16384
N_COLS = 50
N_PAIRS = N_ROWS * N_COLS
C = 128
BLOCKS_PER_TILE = N_ROWS // C // NW
NCHUNK = N_COLS * BLOCKS_PER_TILE
GROUPS = C // L

NBUF = 2


def _sc_kernel(idx_hbm, emb_hbm, out_hbm,
               idx_v, rows_a0, rows_a1, rows_b0, rows_b1,
               out_v, stage_v, sem0, sem1, isem0, isem1):
    wid = lax.axis_index("s") * NC + lax.axis_index("c")
    lane = jnp.arange(L, dtype=jnp.int32)
    sems = [sem0, sem1]
    isems = [isem0, isem1]
    rows_as = [rows_a0, rows_a1]
    rows_bs = [rows_b0, rows_b1]

    def chunk_offsets(h):
        c = h // BLOCKS_PER_TILE
        m1 = wid * BLOCKS_PER_TILE + h % BLOCKS_PER_TILE
        return c * (2 * N_ROWS) + m1 * (2 * C), c * N_ROWS + m1 * C

    def idx_async(h, slot):
        idx0, _ = chunk_offsets(h)
        pltpu.async_copy(idx_hbm.at[pl.ds(idx0, 2 * C)],
                         idx_v.at[pl.ds(slot * 2 * C, 2 * C)], isems[slot])

    def idx_wait(slot):
        pltpu.make_async_copy(idx_hbm.at[pl.ds(0, 2 * C)],
                              idx_v.at[pl.ds(slot * 2 * C, 2 * C)],
                              isems[slot]).wait()

    def fire_gathers(b):
        pltpu.async_copy(
            emb_hbm.at[idx_v.at[pl.ds(b * 2 * C, C)]], rows_as[b], sems[b]
        )
        pltpu.async_copy(
            emb_hbm.at[idx_v.at[pl.ds(b * 2 * C + C, C)]], rows_bs[b], sems[b]
        )

    def wait_rows(b):
        pltpu.make_async_copy(emb_hbm.at[pl.ds(0, C)], rows_as[b], sems[b]).wait()
        pltpu.make_async_copy(emb_hbm.at[pl.ds(0, C)], rows_bs[b], sems[b]).wait()

    def compute_chunk(h, b):
        ra = rows_as[b]
        rb = rows_bs[b]

        @pl.loop(0, GROUPS)
        def _group(t):
            for k in range(L):
                p = t * L + k
                z = jnp.zeros((L,), jnp.float32)
                for q in range(DIM // L):
                    a = ra[p, pl.ds(q * L, L)]
                    bb = rb[p, pl.ds(q * L, L)]
                    r = jnp.maximum(a - bb, 0.0)
                    z = z + r * r
                stage_v[pl.ds(k * L, L)] = z
            vec = jnp.zeros((L,), jnp.float32)
            for d in range(L):
                vec = vec + plsc.load_gather(stage_v, [lane * L + d])
            out_v[pl.ds(t * L, L)] = -vec

        _, out0 = chunk_offsets(h)
        pltpu.sync_copy(out_v, out_hbm.at[pl.ds(out0, C)])

    idx0, _ = chunk_offsets(0)
    pltpu.sync_copy(idx_hbm.at[pl.ds(idx0, 2 * C)], idx_v.at[pl.ds(0, 2 * C)])
    fire_gathers(0)
    idx_async(1, 1)

    @pl.loop(0, NCHUNK // NBUF)
    def _outer(gg):
        for b in range(NBUF):
            h = gg * NBUF + b
            nb = (b + 1) % NBUF

            @pl.when(h + 1 < NCHUNK)
            def _fire_next():
                idx_wait(nb)
                fire_gathers(nb)

            wait_rows(b)

            @pl.when(h + 2 < NCHUNK)
            def _prefetch_idx():
                idx_async(h + 2, b)

            compute_chunk(h, b)


@jax.jit
def kernel(idxs, emb):
    idx32 = idxs.astype(jnp.int32)
    idx_flat = (
        idx32.transpose(1, 2, 0)
        .reshape(N_COLS, 2, N_ROWS // C, C)
        .transpose(0, 2, 1, 3)
        .reshape(-1)
    )
    emb_p = jnp.pad(emb, ((0, 0), (0, 128 - DIM)))
    mesh = plsc.VectorSubcoreMesh(
        core_axis_name="c", subcore_axis_name="s", num_cores=NC, num_subcores=NS
    )
    out = pl.kernel(
        _sc_kernel,
        out_type=jax.ShapeDtypeStruct((N_PAIRS,), jnp.float32),
        mesh=mesh,
        scratch_types=[
            pltpu.VMEM((NBUF * 2 * C,), jnp.int32),
            pltpu.VMEM((C, 128), jnp.float32),
            pltpu.VMEM((C, 128), jnp.float32),
            pltpu.VMEM((C, 128), jnp.float32),
            pltpu.VMEM((C, 128), jnp.float32),
            pltpu.VMEM((C,), jnp.float32),
            pltpu.VMEM((L * L,), jnp.float32),
            pltpu.SemaphoreType.DMA,
            pltpu.SemaphoreType.DMA,
            pltpu.SemaphoreType.DMA,
            pltpu.SemaphoreType.DMA,
        ],
        compiler_params=pltpu.CompilerParams(needs_layout_passes=False),
    )(idx_flat, emb_p)
    return out.reshape(N_COLS, N_ROWS).T


# speedup vs baseline: 1.4049x; 1.0179x over previous
import jax
import jax.numpy as jnp
from jax import lax
from jax.experimental import pallas as pl
from jax.experimental.pallas import tpu as pltpu
from jax.experimental.pallas import tpu_sc as plsc

NC = 2
NS = 16
NW = NC * NS
L = 16

DIM = 64
N_ROWS = 16384
N_COLS = 50
N_PAIRS = N_ROWS * N_COLS
C = 128
BLOCKS_PER_TILE = N_ROWS // C // NW
NCHUNK = N_COLS * BLOCKS_PER_TILE
GROUPS = C // L

NBUF = 3


def _sc_kernel(idx_hbm, emb_hbm, out_hbm,
               idx_v, rows_a0, rows_a1, rows_a2, rows_b0, rows_b1, rows_b2,
               out_v, stage_v, sem0, sem1, sem2, isem0, isem1, isem2):
    wid = lax.axis_index("s") * NC + lax.axis_index("c")
    lane = jnp.arange(L, dtype=jnp.int32)
    sems = [sem0, sem1, sem2]
    isems = [isem0, isem1, isem2]
    rows_as = [rows_a0, rows_a1, rows_a2]
    rows_bs = [rows_b0, rows_b1, rows_b2]

    def chunk_offsets(h):
        c = h // BLOCKS_PER_TILE
        m1 = wid * BLOCKS_PER_TILE + h % BLOCKS_PER_TILE
        return c * (2 * N_ROWS) + m1 * (2 * C), c * N_ROWS + m1 * C

    def idx_async(h, slot):
        idx0, _ = chunk_offsets(h)
        pltpu.async_copy(idx_hbm.at[pl.ds(idx0, 2 * C)],
                         idx_v.at[pl.ds(slot * 2 * C, 2 * C)], isems[slot])

    def idx_wait(slot):
        pltpu.make_async_copy(idx_hbm.at[pl.ds(0, 2 * C)],
                              idx_v.at[pl.ds(slot * 2 * C, 2 * C)],
                              isems[slot]).wait()

    def fire_gathers(b):
        pltpu.async_copy(
            emb_hbm.at[idx_v.at[pl.ds(b * 2 * C, C)]], rows_as[b], sems[b]
        )
        pltpu.async_copy(
            emb_hbm.at[idx_v.at[pl.ds(b * 2 * C + C, C)]], rows_bs[b], sems[b]
        )

    def wait_rows(b):
        pltpu.make_async_copy(emb_hbm.at[pl.ds(0, C)], rows_as[b], sems[b]).wait()
        pltpu.make_async_copy(emb_hbm.at[pl.ds(0, C)], rows_bs[b], sems[b]).wait()

    def compute_chunk(h, b):
        ra = rows_as[b]
        rb = rows_bs[b]

        @pl.loop(0, GROUPS)
        def _group(t):
            for k in range(L):
                p = t * L + k
                z = jnp.zeros((L,), jnp.float32)
                for q in range(DIM // L):
                    a = ra[p, pl.ds(q * L, L)]
                    bb = rb[p, pl.ds(q * L, L)]
                    r = jnp.maximum(a - bb, 0.0)
                    z = z + r * r
                stage_v[pl.ds(k * L, L)] = z
            vec = jnp.zeros((L,), jnp.float32)
            for d in range(L):
                vec = vec + plsc.load_gather(stage_v, [lane * L + d])
            out_v[pl.ds(t * L, L)] = -vec

        _, out0 = chunk_offsets(h)
        pltpu.sync_copy(out_v, out_hbm.at[pl.ds(out0, C)])

    for h0 in range(2):
        i0, _ = chunk_offsets(h0)
        pltpu.sync_copy(idx_hbm.at[pl.ds(i0, 2 * C)],
                        idx_v.at[pl.ds(h0 * 2 * C, 2 * C)])
        fire_gathers(h0)
    idx_async(2, 2)

    def body(h, b):
        nb = (b + 2) % NBUF

        @pl.when(h + 2 < NCHUNK)
        def _fire_next():
            idx_wait(nb)
            fire_gathers(nb)

        wait_rows(b)

        @pl.when(h + 3 < NCHUNK)
        def _prefetch_idx():
            idx_async(h + 3, b)

        compute_chunk(h, b)

    MAIN = NCHUNK // NBUF * NBUF

    @pl.loop(0, MAIN // NBUF)
    def _outer(gg):
        for b in range(NBUF):
            body(gg * NBUF + b, b)

    for h in range(MAIN, NCHUNK):
        body(h, h % NBUF)


@jax.jit
def kernel(idxs, emb):
    idx32 = idxs.astype(jnp.int32)
    idx_flat = (
        idx32.transpose(1, 2, 0)
        .reshape(N_COLS, 2, N_ROWS // C, C)
        .transpose(0, 2, 1, 3)
        .reshape(-1)
    )
    emb_p = jnp.pad(emb, ((0, 0), (0, 128 - DIM)))
    mesh = plsc.VectorSubcoreMesh(
        core_axis_name="c", subcore_axis_name="s", num_cores=NC, num_subcores=NS
    )
    out = pl.kernel(
        _sc_kernel,
        out_type=jax.ShapeDtypeStruct((N_PAIRS,), jnp.float32),
        mesh=mesh,
        scratch_types=[
            pltpu.VMEM((NBUF * 2 * C,), jnp.int32),
            pltpu.VMEM((C, 128), jnp.float32),
            pltpu.VMEM((C, 128), jnp.float32),
            pltpu.VMEM((C, 128), jnp.float32),
            pltpu.VMEM((C, 128), jnp.float32),
            pltpu.VMEM((C, 128), jnp.float32),
            pltpu.VMEM((C, 128), jnp.float32),
            pltpu.VMEM((C,), jnp.float32),
            pltpu.VMEM((L * L,), jnp.float32),
            pltpu.SemaphoreType.DMA,
            pltpu.SemaphoreType.DMA,
            pltpu.SemaphoreType.DMA,
            pltpu.SemaphoreType.DMA,
            pltpu.SemaphoreType.DMA,
            pltpu.SemaphoreType.DMA,
        ],
        compiler_params=pltpu.CompilerParams(needs_layout_passes=False),
    )(idx_flat, emb_p)
    return out.reshape(N_COLS, N_ROWS).T


# speedup vs baseline: 1.5710x; 1.1182x over previous
import jax
import jax.numpy as jnp
from jax import lax
from jax.experimental import pallas as pl
from jax.experimental.pallas import tpu as pltpu
from jax.experimental.pallas import tpu_sc as plsc

NC = 2
NS = 16
NW = NC * NS
L = 16

DIM = 64
N_ROWS = 16384
N_COLS = 50
N_PAIRS = N_ROWS * N_COLS
C = 128
BLOCKS_PER_TILE = N_ROWS // C // NW
NCHUNK = N_COLS * BLOCKS_PER_TILE
GROUPS = C // L

NBUF = 3


def _sc_kernel(idx_hbm, emb_hbm, out_hbm,
               idx_v, rows_a0, rows_a1, rows_a2, rows_b0, rows_b1, rows_b2,
               out_v, stage_v, sem0, sem1, sem2, isem0, isem1, isem2):
    wid = lax.axis_index("s") * NC + lax.axis_index("c")
    lane = jnp.arange(L, dtype=jnp.int32)
    sems = [sem0, sem1, sem2]
    isems = [isem0, isem1, isem2]
    rows_as = [rows_a0, rows_a1, rows_a2]
    rows_bs = [rows_b0, rows_b1, rows_b2]

    def chunk_offsets(h):
        c = h // BLOCKS_PER_TILE
        m1 = wid * BLOCKS_PER_TILE + h % BLOCKS_PER_TILE
        return c * (2 * N_ROWS) + m1 * (2 * C), c * N_ROWS + m1 * C

    def idx_async(h, slot):
        idx0, _ = chunk_offsets(h)
        pltpu.async_copy(idx_hbm.at[pl.ds(idx0, 2 * C)],
                         idx_v.at[pl.ds(slot * 2 * C, 2 * C)], isems[slot])

    def idx_wait(slot):
        pltpu.make_async_copy(idx_hbm.at[pl.ds(0, 2 * C)],
                              idx_v.at[pl.ds(slot * 2 * C, 2 * C)],
                              isems[slot]).wait()

    def fire_gathers(b):
        pltpu.async_copy(
            emb_hbm.at[idx_v.at[pl.ds(b * 2 * C, C)]], rows_as[b], sems[b]
        )
        pltpu.async_copy(
            emb_hbm.at[idx_v.at[pl.ds(b * 2 * C + C, C)]], rows_bs[b], sems[b]
        )

    def wait_rows(b):
        pltpu.make_async_copy(emb_hbm.at[pl.ds(0, C)], rows_as[b], sems[b]).wait()
        pltpu.make_async_copy(emb_hbm.at[pl.ds(0, C)], rows_bs[b], sems[b]).wait()

    def compute_chunk(h, b):
        ra = rows_as[b]
        rb = rows_bs[b]

        @pl.loop(0, GROUPS)
        def _group(t):
            zs = []
            for k in range(L):
                p = t * L + k
                z = jnp.zeros((L,), jnp.float32)
                for q in range(DIM // L):
                    a = ra[p, pl.ds(q * L, L)]
                    bb = rb[p, pl.ds(q * L, L)]
                    r = jnp.maximum(a - bb, 0.0)
                    z = z + r * r
                zs.append(z)
            dnums = lax.GatherDimensionNumbers(
                offset_dims=(), collapsed_slice_dims=(0,), start_index_map=(0,)
            )

            def shuffle(v, perm):
                return lax.gather(
                    v, perm[:, None], dimension_numbers=dnums, slice_sizes=(1,),
                    mode=lax.GatherScatterMode.PROMISE_IN_BOUNDS,
                )

            d = 1
            while d < L:
                m = (lane & d) == 0
                perm = lane ^ d
                zs = [
                    jnp.where(m, x, y) + shuffle(jnp.where(m, y, x), perm)
                    for x, y in zip(zs[0::2], zs[1::2])
                ]
                d *= 2
            out_v[pl.ds(t * L, L)] = -zs[0]

        _, out0 = chunk_offsets(h)
        pltpu.sync_copy(out_v, out_hbm.at[pl.ds(out0, C)])

    for h0 in range(2):
        i0, _ = chunk_offsets(h0)
        pltpu.sync_copy(idx_hbm.at[pl.ds(i0, 2 * C)],
                        idx_v.at[pl.ds(h0 * 2 * C, 2 * C)])
        fire_gathers(h0)
    idx_async(2, 2)

    def body(h, b):
        nb = (b + 2) % NBUF

        @pl.when(h + 2 < NCHUNK)
        def _fire_next():
            idx_wait(nb)
            fire_gathers(nb)

        wait_rows(b)

        @pl.when(h + 3 < NCHUNK)
        def _prefetch_idx():
            idx_async(h + 3, b)

        compute_chunk(h, b)

    MAIN = NCHUNK // NBUF * NBUF

    @pl.loop(0, MAIN // NBUF)
    def _outer(gg):
        for b in range(NBUF):
            body(gg * NBUF + b, b)

    for h in range(MAIN, NCHUNK):
        body(h, h % NBUF)


@jax.jit
def kernel(idxs, emb):
    idx32 = idxs.astype(jnp.int32)
    idx_flat = (
        idx32.transpose(1, 2, 0)
        .reshape(N_COLS, 2, N_ROWS // C, C)
        .transpose(0, 2, 1, 3)
        .reshape(-1)
    )
    emb_p = jnp.pad(emb, ((0, 0), (0, 128 - DIM)))
    mesh = plsc.VectorSubcoreMesh(
        core_axis_name="c", subcore_axis_name="s", num_cores=NC, num_subcores=NS
    )
    out = pl.kernel(
        _sc_kernel,
        out_type=jax.ShapeDtypeStruct((N_PAIRS,), jnp.float32),
        mesh=mesh,
        scratch_types=[
            pltpu.VMEM((NBUF * 2 * C,), jnp.int32),
            pltpu.VMEM((C, 128), jnp.float32),
            pltpu.VMEM((C, 128), jnp.float32),
            pltpu.VMEM((C, 128), jnp.float32),
            pltpu.VMEM((C, 128), jnp.float32),
            pltpu.VMEM((C, 128), jnp.float32),
            pltpu.VMEM((C, 128), jnp.float32),
            pltpu.VMEM((C,), jnp.float32),
            pltpu.VMEM((L * L,), jnp.float32),
            pltpu.SemaphoreType.DMA,
            pltpu.SemaphoreType.DMA,
            pltpu.SemaphoreType.DMA,
            pltpu.SemaphoreType.DMA,
            pltpu.SemaphoreType.DMA,
            pltpu.SemaphoreType.DMA,
        ],
        compiler_params=pltpu.CompilerParams(needs_layout_passes=False),
    )(idx_flat, emb_p)
    return out.reshape(N_COLS, N_ROWS).T
